# baseline (speedup 1.0000x reference)
import functools

import jax
import jax.numpy as jnp
from jax import lax
from jax.experimental import pallas as pl
from jax.experimental.pallas import tpu as pltpu
from jax.experimental.pallas import tpu_sc as plsc

AMP_NOISE = 0.5
NBINS = 4
L = 128
D = 32
B = 4096
NTRAIN = 100000

NUM_CORES = 2
NUM_SUBCORES = 16
NLANES = 16
NW = NUM_CORES * NUM_SUBCORES
SPW = B // NW
NG = SPW // NLANES


def _sc_body(xidx_hbm, lab_hbm, conf_hbm, interp_hbm, noise_hbm,
             imin_hbm, imax_hbm, cs_hbm, rmin_hbm, rmax_hbm,
             xv, lv, rows, csv, iminv, imaxv, gminv, gmaxv, nminv, nmaxv,
             sem):
    cid = lax.axis_index("c")
    sid = lax.axis_index("s")
    wid = sid * NUM_CORES + cid
    base = wid * SPW

    pltpu.sync_copy(xidx_hbm.at[pl.ds(base, SPW)], xv)
    pltpu.sync_copy(lab_hbm.at[pl.ds(base, SPW)], lv)
    pltpu.async_copy(interp_hbm.at[xv], rows, sem).wait()
    pltpu.async_copy(conf_hbm.at[xv], csv, sem).wait()

    lane = lax.iota(jnp.int32, NLANES)
    inf16 = jnp.full((NLANES,), jnp.inf, jnp.float32)
    zero16 = jnp.zeros((NLANES,), jnp.int32)

    for g in range(NG):
        s16 = g * NLANES + lane
        lab16 = lv[pl.ds(g * NLANES, NLANES)]
        is0 = lab16 == 0
        is1 = lab16 == 1
        is2 = lab16 == 2

        def lbody(l, carry, s16=s16, is0=is0, is1=is1, is2=is2):
            dlo, ilo, dhi, ihi = carry
            c = zero16 + l
            x0 = plsc.load_gather(rows, [s16, c])
            x1 = plsc.load_gather(rows, [s16, c + L])
            x2 = plsc.load_gather(rows, [s16, c + 2 * L])
            x3 = plsc.load_gather(rows, [s16, c + 3 * L])
            xs = jnp.where(is0, x0, jnp.where(is1, x1, jnp.where(is2, x2, x3)))
            d = (jnp.exp(x0 - xs) + jnp.exp(x1 - xs)
                 + jnp.exp(x2 - xs) + jnp.exp(x3 - xs))
            mlt = d < dlo
            dlo = jnp.where(mlt, d, dlo)
            ilo = jnp.where(mlt, l, ilo)
            mgt = d > dhi
            dhi = jnp.where(mgt, d, dhi)
            ihi = jnp.where(mgt, l, ihi)
            return dlo, ilo, dhi, ihi

        _, ilo, _, ihi = lax.fori_loop(
            0, L, lbody, (inf16, zero16, -inf16, zero16))
        imaxv[pl.ds(g * NLANES, NLANES)] = ilo.astype(jnp.float32)
        iminv[pl.ds(g * NLANES, NLANES)] = ihi.astype(jnp.float32)
        rowbase = (base + s16) * L
        gmaxv[pl.ds(g * NLANES, NLANES)] = rowbase + ilo
        gminv[pl.ds(g * NLANES, NLANES)] = rowbase + ihi

    pltpu.async_copy(noise_hbm.at[gminv], nminv, sem).wait()
    pltpu.async_copy(noise_hbm.at[gmaxv], nmaxv, sem).wait()

    pltpu.sync_copy(iminv, imin_hbm.at[pl.ds(base, SPW)])
    pltpu.sync_copy(imaxv, imax_hbm.at[pl.ds(base, SPW)])
    pltpu.sync_copy(csv, cs_hbm.at[pl.ds(base, SPW)])
    pltpu.sync_copy(nminv, rmin_hbm.at[pl.ds(base, SPW)])
    pltpu.sync_copy(nmaxv, rmax_hbm.at[pl.ds(base, SPW)])


_sc_stage = functools.partial(
    pl.kernel,
    out_type=[
        jax.ShapeDtypeStruct((B,), jnp.float32),
        jax.ShapeDtypeStruct((B,), jnp.float32),
        jax.ShapeDtypeStruct((B,), jnp.float32),
        jax.ShapeDtypeStruct((B, D), jnp.float32),
        jax.ShapeDtypeStruct((B, D), jnp.float32),
    ],
    mesh=plsc.VectorSubcoreMesh(
        core_axis_name="c", subcore_axis_name="s",
        num_cores=NUM_CORES, num_subcores=NUM_SUBCORES),
    scratch_types=[
        pltpu.VMEM((SPW,), jnp.int32),
        pltpu.VMEM((SPW,), jnp.int32),
        pltpu.VMEM((SPW, NBINS * L), jnp.float32),
        pltpu.VMEM((SPW,), jnp.float32),
        pltpu.VMEM((SPW,), jnp.float32),
        pltpu.VMEM((SPW,), jnp.float32),
        pltpu.VMEM((SPW,), jnp.int32),
        pltpu.VMEM((SPW,), jnp.int32),
        pltpu.VMEM((SPW, D), jnp.float32),
        pltpu.VMEM((SPW, D), jnp.float32),
        pltpu.SemaphoreType.DMA,
    ],
    compiler_params=pltpu.CompilerParams(
        use_tc_tiling_on_sc=False, needs_layout_passes=False),
)(_sc_body)


_TC_BS = 128


def _tc_body(patch_ref, imin_ref, imax_ref, cs_ref, rmin_ref, rmax_ref,
             out_ref):
    p = patch_ref[...]
    s1 = jnp.sum(p, axis=-1, keepdims=True)
    s2 = jnp.sum(p * p, axis=-1, keepdims=True)
    var = (s2 - s1 * s1 * (1.0 / D)) * (1.0 / (D - 1))
    sd = jnp.sqrt(var)
    bs = _TC_BS
    imin = imin_ref[...].astype(jnp.int32).reshape(bs, 1, 1, 1)
    imax = imax_ref[...].astype(jnp.int32).reshape(bs, 1, 1, 1)
    cs = cs_ref[...].reshape(bs, 1, 1, 1)
    pos = lax.broadcasted_iota(jnp.int32, (bs, 1, L, 1), 2)
    cmin = jnp.where(pos == imin, cs * AMP_NOISE * sd, 0.0)
    cmax = jnp.where(jnp.logical_and(pos == imax, imin != imax),
                     (1.0 - cs) * AMP_NOISE * sd, 0.0)
    rmin_b = rmin_ref[...].reshape(bs, 1, 1, D)
    rmax_b = rmax_ref[...].reshape(bs, 1, 1, D)
    out_ref[...] = p + cmin * rmin_b + cmax * rmax_b


def _tc_stage(patch, imin, imax, cs, rmin, rmax):
    bs = _TC_BS
    grid = (B // bs,)
    return pl.pallas_call(
        _tc_body,
        grid=grid,
        in_specs=[
            pl.BlockSpec((bs, 1, L, D), lambda i: (i, 0, 0, 0)),
            pl.BlockSpec((bs,), lambda i: (i,)),
            pl.BlockSpec((bs,), lambda i: (i,)),
            pl.BlockSpec((bs,), lambda i: (i,)),
            pl.BlockSpec((bs, D), lambda i: (i, 0)),
            pl.BlockSpec((bs, D), lambda i: (i, 0)),
        ],
        out_specs=pl.BlockSpec((bs, 1, L, D), lambda i: (i, 0, 0, 0)),
        out_shape=jax.ShapeDtypeStruct((B, 1, L, D), jnp.float32),
    )(patch, imin, imax, cs, rmin, rmax)


def kernel(patch, noise_base, labels, x_idx, conf_score, interpre):
    x_idx = x_idx.astype(jnp.int32)
    labels = labels.astype(jnp.int32)
    interp2 = interpre.reshape(NTRAIN, NBINS * L)
    noise2 = noise_base.reshape(B * L, D)
    imin, imax, cs, rmin, rmax = _sc_stage(
        x_idx, labels, conf_score, interp2, noise2)
    return _tc_stage(patch, imin, imax, cs, rmin, rmax)
